# baseline (speedup 1.0000x reference)
import functools

import jax
import jax.numpy as jnp
from jax import lax
from jax.experimental import pallas as pl
from jax.experimental.pallas import tpu as pltpu
from jax.experimental.pallas import tpu_sc as plsc

NC = 2
NS = 16
NW = NC * NS
LANES = 16
R_TC = 512
W_SC = R_TC // 2

NEG = float("-inf")


def _sc_stage(flat1d, starts, ends, offs, d, b):
    mesh = plsc.VectorSubcoreMesh(
        core_axis_name="c", subcore_axis_name="s", num_cores=NC, num_subcores=NS
    )

    @functools.partial(
        pl.kernel,
        out_type=jax.ShapeDtypeStruct((NW * b * d,), jnp.float32),
        mesh=mesh,
        scratch_types=[
            pltpu.VMEM((W_SC * d,), jnp.float32),
            pltpu.VMEM((b,), jnp.int32),
            pltpu.VMEM((b,), jnp.int32),
            pltpu.VMEM((LANES,), jnp.int32),
            pltpu.VMEM((b * d,), jnp.float32),
            pltpu.SemaphoreType.DMA,
        ],
    )
    def k(flat_hbm, st_hbm, en_hbm, off_hbm, out_hbm, buf, st_v, en_v, off_v, acc_v, sem):
        cid = lax.axis_index("c")
        sid = lax.axis_index("s")
        wid = sid * NC + cid

        pltpu.sync_copy(off_hbm.at[pl.ds(wid * LANES, LANES)], off_v)
        base = pl.multiple_of(off_v[...][0], W_SC * d)
        pltpu.make_async_copy(flat_hbm.at[pl.ds(base, W_SC * d)], buf, sem).start()

        pltpu.sync_copy(st_hbm.at[pl.ds(wid * b, b)], st_v)
        pltpu.sync_copy(en_hbm.at[pl.ds(wid * b, b)], en_v)
        st_vec = st_v[...]
        en_vec = en_v[...]

        neg = jnp.full((LANES,), NEG, jnp.float32)
        for kk in range(b * d // LANES):
            acc_v[pl.ds(kk * LANES, LANES)] = neg

        pltpu.make_async_copy(flat_hbm.at[pl.ds(base, W_SC * d)], buf, sem).wait()
        for s in range(b):
            lo = st_vec[s]
            hi = en_vec[s]
            accs = tuple(
                acc_v[pl.ds(s * d + LANES * j, LANES)] for j in range(d // LANES)
            )

            def rbody(r, a):
                off = r * d
                return tuple(
                    jnp.maximum(aj, buf[pl.ds(off + LANES * j, LANES)])
                    for j, aj in enumerate(a)
                )

            accs = plsc.parallel_loop(lo, hi, unroll=4, carry=accs)(rbody)
            for j in range(d // LANES):
                acc_v[pl.ds(s * d + LANES * j, LANES)] = accs[j]

        pltpu.sync_copy(acc_v, out_hbm.at[pl.ds(wid * b * d, b * d)])

    return k(flat1d, starts, ends, offs)


def _tc_blocks(flat, n, d):
    nblk = n // R_TC

    def body(x_ref, o_ref):
        o_ref[...] = jnp.max(x_ref[...], axis=0, keepdims=True)[None]

    return pl.pallas_call(
        body,
        grid=(nblk,),
        in_specs=[pl.BlockSpec((R_TC, d), lambda i: (i, 0))],
        out_specs=pl.BlockSpec((1, 1, d), lambda i: (i, 0, 0)),
        out_shape=jax.ShapeDtypeStruct((nblk, 1, d), jnp.float32),
    )(flat)


def _tc_merge(partials_sc, blockmax, segmap, b, d, nblk):
    def body(seg_ref, q_ref, p_ref, o_ref):
        segid = lax.broadcasted_iota(jnp.int32, (b, 1), 0)
        acc = jnp.full((b, d), NEG, jnp.float32)
        for w in range(NW):
            acc = jnp.maximum(acc, p_ref[w * b : (w + 1) * b, :])
        for i in range(nblk):
            seg = seg_ref[i]
            upd = jnp.maximum(acc, q_ref[i : i + 1, :])
            acc = jnp.where(segid == seg, upd, acc)
        o_ref[...] = acc

    return pl.pallas_call(
        body,
        in_specs=[
            pl.BlockSpec(memory_space=pltpu.SMEM),
            pl.BlockSpec((nblk, d), lambda: (0, 0)),
            pl.BlockSpec((NW * b, d), lambda: (0, 0)),
        ],
        out_specs=pl.BlockSpec((b, d), lambda: (0, 0)),
        out_shape=jax.ShapeDtypeStruct((b, d), jnp.float32),
    )(segmap, blockmax, partials_sc)


def kernel(flat, cu_seqlens):
    n, d = flat.shape
    b = cu_seqlens.shape[0] - 1
    nblk = n // R_TC
    assert n % R_TC == 0 and d % LANES == 0

    cu = cu_seqlens.astype(jnp.int32)

    r0 = jnp.arange(nblk, dtype=jnp.int32) * R_TC
    inner = cu[1:b][None, :]
    s_first = jnp.sum(inner <= r0[:, None], axis=1, dtype=jnp.int32)
    s_last = jnp.sum(inner <= (r0 + R_TC - 1)[:, None], axis=1, dtype=jnp.int32)
    segmap = jnp.where(s_first == s_last, s_first, jnp.int32(b))

    w = jnp.arange(NW, dtype=jnp.int32)
    t = jnp.minimum(w // 2 + 1, b - 1)
    win = (cu[t] // R_TC) * R_TC
    wbase = win + (w % 2) * W_SC
    starts = jnp.clip(cu[None, :-1], wbase[:, None], wbase[:, None] + W_SC)
    ends = jnp.clip(cu[None, 1:], wbase[:, None], wbase[:, None] + W_SC)
    starts = (starts - wbase[:, None]).reshape(-1)
    ends = (ends - wbase[:, None]).reshape(-1)
    offs = jnp.repeat(wbase * d, LANES)

    partials_sc = _sc_stage(flat.reshape(-1), starts, ends, offs, d, b)
    blockmax = _tc_blocks(flat, n, d)
    return _tc_merge(
        partials_sc.reshape(NW * b, d), blockmax.reshape(nblk, d), segmap, b, d, nblk
    )


# speedup vs baseline: 1.8294x; 1.8294x over previous
import functools

import jax
import jax.numpy as jnp
from jax import lax
from jax.experimental import pallas as pl
from jax.experimental.pallas import tpu as pltpu
from jax.experimental.pallas import tpu_sc as plsc

NC = 2
NS = 16
NW = NC * NS
LANES = 16
GRAN = 512
R_TC = 4096
SUB = R_TC // GRAN
W_SC = GRAN // 2

NEG = float("-inf")


def _sc_stage(flat1d, starts, ends, offs, d, b):
    mesh = plsc.VectorSubcoreMesh(
        core_axis_name="c", subcore_axis_name="s", num_cores=NC, num_subcores=NS
    )

    @functools.partial(
        pl.kernel,
        out_type=jax.ShapeDtypeStruct((NW * b * d,), jnp.float32),
        mesh=mesh,
        scratch_types=[
            pltpu.VMEM((W_SC * d,), jnp.float32),
            pltpu.VMEM((b,), jnp.int32),
            pltpu.VMEM((b,), jnp.int32),
            pltpu.VMEM((LANES,), jnp.int32),
            pltpu.VMEM((b * d,), jnp.float32),
            pltpu.SemaphoreType.DMA,
        ],
    )
    def k(flat_hbm, st_hbm, en_hbm, off_hbm, out_hbm, buf, st_v, en_v, off_v, acc_v, sem):
        cid = lax.axis_index("c")
        sid = lax.axis_index("s")
        wid = sid * NC + cid

        pltpu.sync_copy(off_hbm.at[pl.ds(wid * LANES, LANES)], off_v)
        base = pl.multiple_of(off_v[...][0], W_SC * d)
        pltpu.make_async_copy(flat_hbm.at[pl.ds(base, W_SC * d)], buf, sem).start()

        pltpu.sync_copy(st_hbm.at[pl.ds(wid * b, b)], st_v)
        pltpu.sync_copy(en_hbm.at[pl.ds(wid * b, b)], en_v)
        st_vec = st_v[...]
        en_vec = en_v[...]

        neg = jnp.full((LANES,), NEG, jnp.float32)
        for kk in range(b * d // LANES):
            acc_v[pl.ds(kk * LANES, LANES)] = neg

        pltpu.make_async_copy(flat_hbm.at[pl.ds(base, W_SC * d)], buf, sem).wait()
        for s in range(b):
            lo = st_vec[s]
            hi = en_vec[s]
            accs = tuple(
                acc_v[pl.ds(s * d + LANES * j, LANES)] for j in range(d // LANES)
            )

            def rbody(r, a):
                off = r * d
                return tuple(
                    jnp.maximum(aj, buf[pl.ds(off + LANES * j, LANES)])
                    for j, aj in enumerate(a)
                )

            accs = plsc.parallel_loop(lo, hi, unroll=4, carry=accs)(rbody)
            for j in range(d // LANES):
                acc_v[pl.ds(s * d + LANES * j, LANES)] = accs[j]

        pltpu.sync_copy(acc_v, out_hbm.at[pl.ds(wid * b * d, b * d)])

    return k(flat1d, starts, ends, offs)


def _tc_blocks(flat, n, d):
    nstep = n // R_TC

    def body(x_ref, o_ref):
        x = x_ref[...]
        subs = [
            jnp.max(x[j * GRAN : (j + 1) * GRAN], axis=0, keepdims=True)
            for j in range(SUB)
        ]
        o_ref[...] = jnp.concatenate(subs, axis=0)[None]

    return pl.pallas_call(
        body,
        grid=(nstep,),
        in_specs=[pl.BlockSpec((R_TC, d), lambda i: (i, 0))],
        out_specs=pl.BlockSpec((1, SUB, d), lambda i: (i, 0, 0)),
        out_shape=jax.ShapeDtypeStruct((nstep, SUB, d), jnp.float32),
    )(flat)


def _tc_merge(partials_sc, blockmax, segmap, b, d, nblk):
    def body(seg_ref, q_ref, p_ref, o_ref):
        segid = lax.broadcasted_iota(jnp.int32, (b, 1), 0)
        acc = jnp.full((b, d), NEG, jnp.float32)
        for w in range(NW):
            acc = jnp.maximum(acc, p_ref[w * b : (w + 1) * b, :])
        for i in range(nblk):
            seg = seg_ref[i]
            upd = jnp.maximum(acc, q_ref[i : i + 1, :])
            acc = jnp.where(segid == seg, upd, acc)
        o_ref[...] = acc

    return pl.pallas_call(
        body,
        in_specs=[
            pl.BlockSpec(memory_space=pltpu.SMEM),
            pl.BlockSpec((nblk, d), lambda: (0, 0)),
            pl.BlockSpec((NW * b, d), lambda: (0, 0)),
        ],
        out_specs=pl.BlockSpec((b, d), lambda: (0, 0)),
        out_shape=jax.ShapeDtypeStruct((b, d), jnp.float32),
    )(segmap, blockmax, partials_sc)


def kernel(flat, cu_seqlens):
    n, d = flat.shape
    b = cu_seqlens.shape[0] - 1
    nblk = n // GRAN
    assert n % R_TC == 0 and d % LANES == 0

    cu = cu_seqlens.astype(jnp.int32)

    r0 = jnp.arange(nblk, dtype=jnp.int32) * GRAN
    inner = cu[1:b][None, :]
    s_first = jnp.sum(inner <= r0[:, None], axis=1, dtype=jnp.int32)
    s_last = jnp.sum(inner <= (r0 + GRAN - 1)[:, None], axis=1, dtype=jnp.int32)
    segmap = jnp.where(s_first == s_last, s_first, jnp.int32(b))

    w = jnp.arange(NW, dtype=jnp.int32)
    t = jnp.minimum(w // 2 + 1, b - 1)
    win = (cu[t] // GRAN) * GRAN
    wbase = win + (w % 2) * W_SC
    starts = jnp.clip(cu[None, :-1], wbase[:, None], wbase[:, None] + W_SC)
    ends = jnp.clip(cu[None, 1:], wbase[:, None], wbase[:, None] + W_SC)
    starts = (starts - wbase[:, None]).reshape(-1)
    ends = (ends - wbase[:, None]).reshape(-1)
    offs = jnp.repeat(wbase * d, LANES)

    partials_sc = _sc_stage(flat.reshape(-1), starts, ends, offs, d, b)
    blockmax = _tc_blocks(flat, n, d)
    return _tc_merge(
        partials_sc.reshape(NW * b, d), blockmax.reshape(nblk, d), segmap, b, d, nblk
    )


# speedup vs baseline: 1.9040x; 1.0408x over previous
import functools

import jax
import jax.numpy as jnp
from jax import lax
from jax.experimental import pallas as pl
from jax.experimental.pallas import tpu as pltpu
from jax.experimental.pallas import tpu_sc as plsc

NC = 2
NS = 16
NW = NC * NS
LANES = 16
GRAN = 512
R_TC = 4096
NSTREAM = 2
SUB = R_TC // GRAN
W_SC = GRAN // 2

NEG = float("-inf")


def _sc_stage(flat1d, cu_lo, cu_hi, cu_hi_rep, d, b):
    mesh = plsc.VectorSubcoreMesh(
        core_axis_name="c", subcore_axis_name="s", num_cores=NC, num_subcores=NS
    )
    nj = d // LANES

    @functools.partial(
        pl.kernel,
        out_type=jax.ShapeDtypeStruct((NW * b * d,), jnp.float32),
        mesh=mesh,
        scratch_types=[
            pltpu.VMEM((W_SC * d,), jnp.float32),
            pltpu.VMEM((b,), jnp.int32),
            pltpu.VMEM((b,), jnp.int32),
            pltpu.VMEM((LANES,), jnp.int32),
            pltpu.VMEM((b * d,), jnp.float32),
            pltpu.SemaphoreType.DMA,
        ],
    )
    def k(flat_hbm, lo_hbm, hi_hbm, rep_hbm, out_hbm, buf, lo_v, hi_v, ct_v, acc_v, sem):
        cid = lax.axis_index("c")
        sid = lax.axis_index("s")
        wid = sid * NC + cid

        t1 = jnp.minimum(wid // 2, b - 2)
        pltpu.sync_copy(rep_hbm.at[pl.ds(t1 * LANES, LANES)], ct_v)
        ct = ct_v[...][0]
        win = (ct // GRAN) * GRAN
        wbase = win + (wid % 2) * W_SC
        base = pl.multiple_of(wbase * d, W_SC * d)
        pltpu.make_async_copy(flat_hbm.at[pl.ds(base, W_SC * d)], buf, sem).start()

        pltpu.sync_copy(lo_hbm.at[pl.ds(0, b)], lo_v)
        pltpu.sync_copy(hi_hbm.at[pl.ds(0, b)], hi_v)
        st_vec = jnp.clip(lo_v[...] - wbase, 0, W_SC)
        en_vec = jnp.clip(hi_v[...] - wbase, 0, W_SC)

        neg = jnp.full((LANES,), NEG, jnp.float32)
        for kk in range(b * nj):
            acc_v[pl.ds(kk * LANES, LANES)] = neg

        pltpu.make_async_copy(flat_hbm.at[pl.ds(base, W_SC * d)], buf, sem).wait()

        for s in range(b):
            lo = st_vec[s]
            hi = en_vec[s]
            accs = tuple(
                acc_v[pl.ds(s * d + LANES * j, LANES)] for j in range(nj)
            )

            def rbody(r, a):
                off = r * d
                return tuple(
                    jnp.maximum(aj, buf[pl.ds(off + LANES * j, LANES)])
                    for j, aj in enumerate(a)
                )

            accs = plsc.parallel_loop(lo, hi, unroll=4, carry=accs)(rbody)
            for j in range(nj):
                acc_v[pl.ds(s * d + LANES * j, LANES)] = accs[j]

        pltpu.sync_copy(acc_v, out_hbm.at[pl.ds(wid * b * d, b * d)])

    return k(flat1d, cu_lo, cu_hi, cu_hi_rep)


def _tc_blocks(flat, n, d):
    nstep = n // (R_TC * NSTREAM)

    def body(xa_ref, xb_ref, o_ref):
        outs = []
        for x_ref in (xa_ref, xb_ref):
            x = x_ref[...]
            outs += [
                jnp.max(x[j * GRAN : (j + 1) * GRAN], axis=0, keepdims=True)
                for j in range(SUB)
            ]
        o_ref[...] = jnp.concatenate(outs, axis=0)[None]

    return pl.pallas_call(
        body,
        grid=(nstep,),
        in_specs=[
            pl.BlockSpec((R_TC, d), lambda i: (NSTREAM * i, 0)),
            pl.BlockSpec((R_TC, d), lambda i: (NSTREAM * i + 1, 0)),
        ],
        out_specs=pl.BlockSpec((1, NSTREAM * SUB, d), lambda i: (i, 0, 0)),
        out_shape=jax.ShapeDtypeStruct((nstep, NSTREAM * SUB, d), jnp.float32),
    )(flat, flat)


def _tc_merge(partials_sc, blockmax, segmap, b, d, nblk):
    def body(seg_ref, q_ref, p_ref, o_ref):
        segid = lax.broadcasted_iota(jnp.int32, (b, 1), 0)
        acc = jnp.full((b, d), NEG, jnp.float32)
        for w in range(NW):
            acc = jnp.maximum(acc, p_ref[w * b : (w + 1) * b, :])
        for i in range(nblk):
            seg = seg_ref[i]
            upd = jnp.maximum(acc, q_ref[i : i + 1, :])
            acc = jnp.where(segid == seg, upd, acc)
        o_ref[...] = acc

    return pl.pallas_call(
        body,
        in_specs=[
            pl.BlockSpec(memory_space=pltpu.SMEM),
            pl.BlockSpec((nblk, d), lambda: (0, 0)),
            pl.BlockSpec((NW * b, d), lambda: (0, 0)),
        ],
        out_specs=pl.BlockSpec((b, d), lambda: (0, 0)),
        out_shape=jax.ShapeDtypeStruct((b, d), jnp.float32),
    )(segmap, blockmax, partials_sc)


def kernel(flat, cu_seqlens):
    n, d = flat.shape
    b = cu_seqlens.shape[0] - 1
    nblk = n // GRAN
    assert n % (R_TC * NSTREAM) == 0 and d % LANES == 0

    cu = cu_seqlens.astype(jnp.int32)

    r0 = jnp.arange(nblk, dtype=jnp.int32) * GRAN
    inner = cu[1:b][None, :]
    s_first = jnp.sum(inner <= r0[:, None], axis=1, dtype=jnp.int32)
    s_last = jnp.sum(inner <= (r0 + GRAN - 1)[:, None], axis=1, dtype=jnp.int32)
    segmap = jnp.where(s_first == s_last, s_first, jnp.int32(b))

    cu_hi = cu[1:]
    partials_sc = _sc_stage(
        flat.reshape(-1), cu[:-1], cu_hi, jnp.repeat(cu_hi, LANES), d, b
    )
    blockmax = _tc_blocks(flat, n, d)
    return _tc_merge(
        partials_sc.reshape(NW * b, d), blockmax.reshape(nblk, d), segmap, b, d, nblk
    )


# speedup vs baseline: 1.9978x; 1.0492x over previous
import functools

import jax
import jax.numpy as jnp
from jax import lax
from jax.experimental import pallas as pl
from jax.experimental.pallas import tpu as pltpu
from jax.experimental.pallas import tpu_sc as plsc

NC = 2
NS = 16
NW = NC * NS
LANES = 16
GRAN = 256
R_TC = 4096
NSTREAM = 4
SUB = R_TC // GRAN
W_SC = GRAN // 2

NEG = float("-inf")


def _sc_stage(flat, cu_lo, cu_hi, cu_hi_rep, d, b):
    mesh = plsc.VectorSubcoreMesh(
        core_axis_name="c", subcore_axis_name="s", num_cores=NC, num_subcores=NS
    )
    nj = d // LANES

    @functools.partial(
        pl.kernel,
        out_type=jax.ShapeDtypeStruct((NW * b * d,), jnp.float32),
        mesh=mesh,
        scratch_types=[
            pltpu.VMEM((W_SC, d), jnp.float32),
            pltpu.VMEM((b,), jnp.int32),
            pltpu.VMEM((b,), jnp.int32),
            pltpu.VMEM((LANES,), jnp.int32),
            pltpu.VMEM((b * d,), jnp.float32),
            pltpu.SemaphoreType.DMA,
        ],
    )
    def k(flat_hbm, lo_hbm, hi_hbm, rep_hbm, out_hbm, buf, lo_v, hi_v, ct_v, acc_v, sem):
        cid = lax.axis_index("c")
        sid = lax.axis_index("s")
        wid = sid * NC + cid

        t1 = jnp.minimum(wid // 2, b - 2)
        pltpu.sync_copy(rep_hbm.at[pl.ds(t1 * LANES, LANES)], ct_v)
        ct = ct_v[...][0]
        win = (ct // GRAN) * GRAN
        wbase = pl.multiple_of(win + (wid % 2) * W_SC, W_SC)
        pltpu.make_async_copy(flat_hbm.at[pl.ds(wbase, W_SC), :], buf, sem).start()

        pltpu.sync_copy(lo_hbm.at[pl.ds(0, b)], lo_v)
        pltpu.sync_copy(hi_hbm.at[pl.ds(0, b)], hi_v)
        st_vec = jnp.clip(lo_v[...] - wbase, 0, W_SC)
        en_vec = jnp.clip(hi_v[...] - wbase, 0, W_SC)

        neg = jnp.full((LANES,), NEG, jnp.float32)
        for kk in range(b * nj):
            acc_v[pl.ds(kk * LANES, LANES)] = neg

        pltpu.make_async_copy(flat_hbm.at[pl.ds(wbase, W_SC), :], buf, sem).wait()

        for s in range(b):
            lo = st_vec[s]
            hi = en_vec[s]
            accs = tuple(
                acc_v[pl.ds(s * d + LANES * j, LANES)] for j in range(nj)
            )

            def rbody(r, a):
                return tuple(
                    jnp.maximum(aj, buf[r, pl.ds(LANES * j, LANES)])
                    for j, aj in enumerate(a)
                )

            accs = plsc.parallel_loop(lo, hi, unroll=4, carry=accs)(rbody)
            for j in range(nj):
                acc_v[pl.ds(s * d + LANES * j, LANES)] = accs[j]

        pltpu.sync_copy(acc_v, out_hbm.at[pl.ds(wid * b * d, b * d)])

    return k(flat, cu_lo, cu_hi, cu_hi_rep)


def _tc_blocks(flat, n, d):
    nstep = n // (R_TC * NSTREAM)

    def body(*refs):
        o_ref = refs[-1]
        outs = []
        for x_ref in refs[:-1]:
            x = x_ref[...]
            outs += [
                jnp.max(x[j * GRAN : (j + 1) * GRAN], axis=0, keepdims=True)
                for j in range(SUB)
            ]
        o_ref[...] = jnp.concatenate(outs, axis=0)[None]

    return pl.pallas_call(
        body,
        grid=(nstep,),
        in_specs=[
            pl.BlockSpec((R_TC, d), functools.partial(lambda k, i: (NSTREAM * i + k, 0), k))
            for k in range(NSTREAM)
        ],
        out_specs=pl.BlockSpec((1, NSTREAM * SUB, d), lambda i: (i, 0, 0)),
        out_shape=jax.ShapeDtypeStruct((nstep, NSTREAM * SUB, d), jnp.float32),
    )(*([flat] * NSTREAM))


def _tc_merge(partials_sc, blockmax, segmap, b, d, nblk):
    def body(seg_ref, q_ref, p_ref, o_ref):
        segid = lax.broadcasted_iota(jnp.int32, (b, 1), 0)
        acc = jnp.full((b, d), NEG, jnp.float32)
        for w in range(NW):
            acc = jnp.maximum(acc, p_ref[w * b : (w + 1) * b, :])
        for i in range(nblk):
            seg = seg_ref[i]
            upd = jnp.maximum(acc, q_ref[i : i + 1, :])
            acc = jnp.where(segid == seg, upd, acc)
        o_ref[...] = acc

    return pl.pallas_call(
        body,
        in_specs=[
            pl.BlockSpec(memory_space=pltpu.SMEM),
            pl.BlockSpec((nblk, d), lambda: (0, 0)),
            pl.BlockSpec((NW * b, d), lambda: (0, 0)),
        ],
        out_specs=pl.BlockSpec((b, d), lambda: (0, 0)),
        out_shape=jax.ShapeDtypeStruct((b, d), jnp.float32),
    )(segmap, blockmax, partials_sc)


def kernel(flat, cu_seqlens):
    n, d = flat.shape
    b = cu_seqlens.shape[0] - 1
    nblk = n // GRAN
    assert n % (R_TC * NSTREAM) == 0 and d % LANES == 0

    cu = cu_seqlens.astype(jnp.int32)

    r0 = jnp.arange(nblk, dtype=jnp.int32) * GRAN
    inner = cu[1:b][None, :]
    s_first = jnp.sum(inner <= r0[:, None], axis=1, dtype=jnp.int32)
    s_last = jnp.sum(inner <= (r0 + GRAN - 1)[:, None], axis=1, dtype=jnp.int32)
    segmap = jnp.where(s_first == s_last, s_first, jnp.int32(b))

    cu_hi = cu[1:]
    partials_sc = _sc_stage(
        flat, cu[:-1], cu_hi, jnp.repeat(cu_hi, LANES), d, b
    )
    blockmax = _tc_blocks(flat, n, d)
    return _tc_merge(
        partials_sc.reshape(NW * b, d), blockmax.reshape(nblk, d), segmap, b, d, nblk
    )


# speedup vs baseline: 2.0331x; 1.0177x over previous
import functools

import jax
import jax.numpy as jnp
from jax import lax
from jax.experimental import pallas as pl
from jax.experimental.pallas import tpu as pltpu
from jax.experimental.pallas import tpu_sc as plsc

NC = 2
NS = 16
NW = NC * NS
LANES = 16
GRAN = 256
R_TC = 4096
NSTREAM = 4
SUB = R_TC // GRAN
W_SC = GRAN // 2

NEG = float("-inf")


def _sc_stage(flat, pk, d, b):
    mesh = plsc.VectorSubcoreMesh(
        core_axis_name="c", subcore_axis_name="s", num_cores=NC, num_subcores=NS
    )
    nj = d // LANES

    @functools.partial(
        pl.kernel,
        out_type=jax.ShapeDtypeStruct((NW * b * d,), jnp.float32),
        mesh=mesh,
        scratch_types=[
            pltpu.VMEM((W_SC, d), jnp.float32),
            pltpu.VMEM((b,), jnp.int32),
            pltpu.VMEM((b,), jnp.int32),
            pltpu.VMEM((LANES,), jnp.int32),
            pltpu.VMEM((b * d,), jnp.float32),
            pltpu.SemaphoreType.DMA,
        ],
    )
    def k(flat_hbm, pk_hbm, out_hbm, buf, lo_v, hi_v, ct_v, acc_v, sem):
        cid = lax.axis_index("c")
        sid = lax.axis_index("s")
        wid = sid * NC + cid

        t1 = jnp.minimum(wid // 2, b - 2)
        pltpu.sync_copy(pk_hbm.at[pl.ds(2 * b + t1 * LANES, LANES)], ct_v)
        ct = ct_v[...][0]
        win = (ct // GRAN) * GRAN
        wbase = pl.multiple_of(win + (wid % 2) * W_SC, W_SC)
        pltpu.make_async_copy(flat_hbm.at[pl.ds(wbase, W_SC), :], buf, sem).start()

        pltpu.sync_copy(pk_hbm.at[pl.ds(0, b)], lo_v)
        pltpu.sync_copy(pk_hbm.at[pl.ds(b, b)], hi_v)
        st_vec = jnp.clip(lo_v[...] - wbase, 0, W_SC)
        en_vec = jnp.clip(hi_v[...] - wbase, 0, W_SC)

        neg = jnp.full((LANES,), NEG, jnp.float32)
        for kk in range(b * nj):
            acc_v[pl.ds(kk * LANES, LANES)] = neg

        pltpu.make_async_copy(flat_hbm.at[pl.ds(wbase, W_SC), :], buf, sem).wait()

        for s in range(b):
            lo = st_vec[s]
            hi = en_vec[s]
            accs = tuple(
                acc_v[pl.ds(s * d + LANES * j, LANES)] for j in range(nj)
            )

            def rbody(r, a):
                return tuple(
                    jnp.maximum(aj, buf[r, pl.ds(LANES * j, LANES)])
                    for j, aj in enumerate(a)
                )

            accs = plsc.parallel_loop(lo, hi, unroll=4, carry=accs)(rbody)
            for j in range(nj):
                acc_v[pl.ds(s * d + LANES * j, LANES)] = accs[j]

        pltpu.sync_copy(acc_v, out_hbm.at[pl.ds(wid * b * d, b * d)])

    return k(flat, pk)


def _tc_blocks(flat, n, d):
    nstep = n // (R_TC * NSTREAM)

    def body(*refs):
        o_ref = refs[-1]
        outs = []
        for x_ref in refs[:-1]:
            x = x_ref[...]
            outs += [
                jnp.max(x[j * GRAN : (j + 1) * GRAN], axis=0, keepdims=True)
                for j in range(SUB)
            ]
        o_ref[...] = jnp.concatenate(outs, axis=0)[None]

    return pl.pallas_call(
        body,
        grid=(nstep,),
        in_specs=[
            pl.BlockSpec((R_TC, d), functools.partial(lambda k, i: (NSTREAM * i + k, 0), k))
            for k in range(NSTREAM)
        ],
        out_specs=pl.BlockSpec((1, NSTREAM * SUB, d), lambda i: (i, 0, 0)),
        out_shape=jax.ShapeDtypeStruct((nstep, NSTREAM * SUB, d), jnp.float32),
    )(*([flat] * NSTREAM))


def _tc_merge(partials_sc, blockmax, segmap, b, d, nblk):
    def body(seg_ref, q_ref, p_ref, o_ref):
        segid = lax.broadcasted_iota(jnp.int32, (b, 1), 0)
        acc = jnp.full((b, d), NEG, jnp.float32)
        for w in range(NW):
            acc = jnp.maximum(acc, p_ref[w * b : (w + 1) * b, :])
        for i in range(nblk):
            seg = seg_ref[i]
            upd = jnp.maximum(acc, q_ref[i : i + 1, :])
            acc = jnp.where(segid == seg, upd, acc)
        o_ref[...] = acc

    return pl.pallas_call(
        body,
        in_specs=[
            pl.BlockSpec(memory_space=pltpu.SMEM),
            pl.BlockSpec((nblk, d), lambda: (0, 0)),
            pl.BlockSpec((NW * b, d), lambda: (0, 0)),
        ],
        out_specs=pl.BlockSpec((b, d), lambda: (0, 0)),
        out_shape=jax.ShapeDtypeStruct((b, d), jnp.float32),
    )(segmap, blockmax, partials_sc)


def kernel(flat, cu_seqlens):
    n, d = flat.shape
    b = cu_seqlens.shape[0] - 1
    nblk = n // GRAN
    assert n % (R_TC * NSTREAM) == 0 and d % LANES == 0

    cu = cu_seqlens.astype(jnp.int32)

    r0 = jnp.arange(nblk, dtype=jnp.int32) * GRAN
    inner = cu[1:b][None, :]
    s_first = jnp.sum(inner <= r0[:, None], axis=1, dtype=jnp.int32)
    s_last = jnp.sum(inner <= (r0 + GRAN - 1)[:, None], axis=1, dtype=jnp.int32)
    segmap = jnp.where(s_first == s_last, s_first, jnp.int32(b))

    cu_hi = cu[1:]
    pk = jnp.concatenate([cu[:b], cu_hi, jnp.repeat(cu_hi[: b - 1], LANES)])
    partials_sc = _sc_stage(flat, pk, d, b)
    blockmax = _tc_blocks(flat, n, d)
    return _tc_merge(
        partials_sc.reshape(NW * b, d), blockmax.reshape(nblk, d), segmap, b, d, nblk
    )


# speedup vs baseline: 2.0506x; 1.0086x over previous
import functools

import jax
import jax.numpy as jnp
from jax import lax
from jax.experimental import pallas as pl
from jax.experimental.pallas import tpu as pltpu
from jax.experimental.pallas import tpu_sc as plsc

NC = 2
NS = 16
NW = NC * NS
LANES = 16
GRAN = 256
R_TC = 2048
NSTREAM = 8
SUB = R_TC // GRAN
W_SC = GRAN // 2

NEG = float("-inf")


def _sc_stage(flat, pk, d, b):
    mesh = plsc.VectorSubcoreMesh(
        core_axis_name="c", subcore_axis_name="s", num_cores=NC, num_subcores=NS
    )
    nj = d // LANES

    @functools.partial(
        pl.kernel,
        out_type=jax.ShapeDtypeStruct((NW * b * d,), jnp.float32),
        mesh=mesh,
        scratch_types=[
            pltpu.VMEM((W_SC, d), jnp.float32),
            pltpu.VMEM((b,), jnp.int32),
            pltpu.VMEM((b,), jnp.int32),
            pltpu.VMEM((LANES,), jnp.int32),
            pltpu.VMEM((b * d,), jnp.float32),
            pltpu.SemaphoreType.DMA,
        ],
    )
    def k(flat_hbm, pk_hbm, out_hbm, buf, lo_v, hi_v, ct_v, acc_v, sem):
        cid = lax.axis_index("c")
        sid = lax.axis_index("s")
        wid = sid * NC + cid

        t1 = jnp.minimum(wid // 2, b - 2)
        pltpu.sync_copy(pk_hbm.at[pl.ds(2 * b + t1 * LANES, LANES)], ct_v)
        ct = ct_v[...][0]
        win = (ct // GRAN) * GRAN
        wbase = pl.multiple_of(win + (wid % 2) * W_SC, W_SC)
        pltpu.make_async_copy(flat_hbm.at[pl.ds(wbase, W_SC), :], buf, sem).start()

        pltpu.sync_copy(pk_hbm.at[pl.ds(0, b)], lo_v)
        pltpu.sync_copy(pk_hbm.at[pl.ds(b, b)], hi_v)
        st_vec = jnp.clip(lo_v[...] - wbase, 0, W_SC)
        en_vec = jnp.clip(hi_v[...] - wbase, 0, W_SC)

        neg = jnp.full((LANES,), NEG, jnp.float32)
        for kk in range(b * nj):
            acc_v[pl.ds(kk * LANES, LANES)] = neg

        pltpu.make_async_copy(flat_hbm.at[pl.ds(wbase, W_SC), :], buf, sem).wait()

        for s in range(b):
            lo = st_vec[s]
            hi = en_vec[s]
            accs = tuple(
                acc_v[pl.ds(s * d + LANES * j, LANES)] for j in range(nj)
            )

            def rbody(r, a):
                return tuple(
                    jnp.maximum(aj, buf[r, pl.ds(LANES * j, LANES)])
                    for j, aj in enumerate(a)
                )

            accs = plsc.parallel_loop(lo, hi, unroll=4, carry=accs)(rbody)
            for j in range(nj):
                acc_v[pl.ds(s * d + LANES * j, LANES)] = accs[j]

        pltpu.sync_copy(acc_v, out_hbm.at[pl.ds(wid * b * d, b * d)])

    return k(flat, pk)


def _tc_blocks(flat, n, d):
    nstep = n // (R_TC * NSTREAM)

    def body(*refs):
        o_ref = refs[-1]
        outs = []
        for x_ref in refs[:-1]:
            x = x_ref[...]
            outs += [
                jnp.max(x[j * GRAN : (j + 1) * GRAN], axis=0, keepdims=True)
                for j in range(SUB)
            ]
        o_ref[...] = jnp.concatenate(outs, axis=0)[None]

    return pl.pallas_call(
        body,
        grid=(nstep,),
        in_specs=[
            pl.BlockSpec((R_TC, d), functools.partial(lambda k, i: (NSTREAM * i + k, 0), k))
            for k in range(NSTREAM)
        ],
        out_specs=pl.BlockSpec((1, NSTREAM * SUB, d), lambda i: (i, 0, 0)),
        out_shape=jax.ShapeDtypeStruct((nstep, NSTREAM * SUB, d), jnp.float32),
    )(*([flat] * NSTREAM))


def _tc_merge(partials_sc, blockmax, segmap, b, d, nblk):
    def body(seg_ref, q_ref, p_ref, o_ref):
        segid = lax.broadcasted_iota(jnp.int32, (b, 1), 0)
        acc = jnp.full((b, d), NEG, jnp.float32)
        for w in range(NW):
            acc = jnp.maximum(acc, p_ref[w * b : (w + 1) * b, :])
        for i in range(nblk):
            seg = seg_ref[i]
            upd = jnp.maximum(acc, q_ref[i : i + 1, :])
            acc = jnp.where(segid == seg, upd, acc)
        o_ref[...] = acc

    return pl.pallas_call(
        body,
        in_specs=[
            pl.BlockSpec(memory_space=pltpu.SMEM),
            pl.BlockSpec((nblk, d), lambda: (0, 0)),
            pl.BlockSpec((NW * b, d), lambda: (0, 0)),
        ],
        out_specs=pl.BlockSpec((b, d), lambda: (0, 0)),
        out_shape=jax.ShapeDtypeStruct((b, d), jnp.float32),
    )(segmap, blockmax, partials_sc)


def kernel(flat, cu_seqlens):
    n, d = flat.shape
    b = cu_seqlens.shape[0] - 1
    nblk = n // GRAN
    assert n % (R_TC * NSTREAM) == 0 and d % LANES == 0

    cu = cu_seqlens.astype(jnp.int32)

    r0 = jnp.arange(nblk, dtype=jnp.int32) * GRAN
    inner = cu[1:b][None, :]
    s_first = jnp.sum(inner <= r0[:, None], axis=1, dtype=jnp.int32)
    s_last = jnp.sum(inner <= (r0 + GRAN - 1)[:, None], axis=1, dtype=jnp.int32)
    segmap = jnp.where(s_first == s_last, s_first, jnp.int32(b))

    cu_hi = cu[1:]
    pk = jnp.concatenate([cu[:b], cu_hi, jnp.repeat(cu_hi[: b - 1], LANES)])
    partials_sc = _sc_stage(flat, pk, d, b)
    blockmax = _tc_blocks(flat, n, d)
    return _tc_merge(
        partials_sc.reshape(NW * b, d), blockmax.reshape(nblk, d), segmap, b, d, nblk
    )


# speedup vs baseline: 2.0805x; 1.0146x over previous
import functools

import jax
import jax.numpy as jnp
from jax import lax
from jax.experimental import pallas as pl
from jax.experimental.pallas import tpu as pltpu
from jax.experimental.pallas import tpu_sc as plsc

NC = 2
NS = 16
NW = NC * NS
LANES = 16
GRAN = 128
R_TC = 2048
NSTREAM = 8
SUB = R_TC // GRAN
W_SC = GRAN // 2

NEG = float("-inf")


def _sc_stage(flat, pk, d, b):
    mesh = plsc.VectorSubcoreMesh(
        core_axis_name="c", subcore_axis_name="s", num_cores=NC, num_subcores=NS
    )
    nj = d // LANES

    @functools.partial(
        pl.kernel,
        out_type=jax.ShapeDtypeStruct((NW * b * d,), jnp.float32),
        mesh=mesh,
        scratch_types=[
            pltpu.VMEM((W_SC, d), jnp.float32),
            pltpu.VMEM((b,), jnp.int32),
            pltpu.VMEM((b,), jnp.int32),
            pltpu.VMEM((LANES,), jnp.int32),
            pltpu.VMEM((b * d,), jnp.float32),
            pltpu.SemaphoreType.DMA,
        ],
    )
    def k(flat_hbm, pk_hbm, out_hbm, buf, lo_v, hi_v, ct_v, acc_v, sem):
        cid = lax.axis_index("c")
        sid = lax.axis_index("s")
        wid = sid * NC + cid

        t1 = jnp.minimum(wid // 2, b - 2)
        pltpu.sync_copy(pk_hbm.at[pl.ds(2 * b + t1 * LANES, LANES)], ct_v)
        ct = ct_v[...][0]
        win = (ct // GRAN) * GRAN
        wbase = pl.multiple_of(win + (wid % 2) * W_SC, W_SC)
        pltpu.make_async_copy(flat_hbm.at[pl.ds(wbase, W_SC), :], buf, sem).start()

        pltpu.sync_copy(pk_hbm.at[pl.ds(0, b)], lo_v)
        pltpu.sync_copy(pk_hbm.at[pl.ds(b, b)], hi_v)
        st_vec = jnp.clip(lo_v[...] - wbase, 0, W_SC)
        en_vec = jnp.clip(hi_v[...] - wbase, 0, W_SC)

        neg = jnp.full((LANES,), NEG, jnp.float32)
        for kk in range(b * nj):
            acc_v[pl.ds(kk * LANES, LANES)] = neg

        pltpu.make_async_copy(flat_hbm.at[pl.ds(wbase, W_SC), :], buf, sem).wait()

        for s in range(b):
            lo = st_vec[s]
            hi = en_vec[s]
            accs = tuple(
                acc_v[pl.ds(s * d + LANES * j, LANES)] for j in range(nj)
            )

            def rbody(r, a):
                return tuple(
                    jnp.maximum(aj, buf[r, pl.ds(LANES * j, LANES)])
                    for j, aj in enumerate(a)
                )

            accs = plsc.parallel_loop(lo, hi, unroll=4, carry=accs)(rbody)
            for j in range(nj):
                acc_v[pl.ds(s * d + LANES * j, LANES)] = accs[j]

        pltpu.sync_copy(acc_v, out_hbm.at[pl.ds(wid * b * d, b * d)])

    return k(flat, pk)


def _tc_blocks(flat, n, d):
    nstep = n // (R_TC * NSTREAM)

    def body(*refs):
        o_ref = refs[-1]
        outs = []
        for x_ref in refs[:-1]:
            x = x_ref[...]
            outs += [
                jnp.max(x[j * GRAN : (j + 1) * GRAN], axis=0, keepdims=True)
                for j in range(SUB)
            ]
        o_ref[...] = jnp.concatenate(outs, axis=0)[None]

    return pl.pallas_call(
        body,
        grid=(nstep,),
        in_specs=[
            pl.BlockSpec((R_TC, d), functools.partial(lambda k, i: (NSTREAM * i + k, 0), k))
            for k in range(NSTREAM)
        ],
        out_specs=pl.BlockSpec((1, NSTREAM * SUB, d), lambda i: (i, 0, 0)),
        out_shape=jax.ShapeDtypeStruct((nstep, NSTREAM * SUB, d), jnp.float32),
    )(*([flat] * NSTREAM))


def _tc_merge(partials_sc, blockmax, segmap, b, d, nblk):
    def body(seg_ref, q_ref, p_ref, o_ref):
        segid = lax.broadcasted_iota(jnp.int32, (b, 1), 0)
        acc = jnp.full((b, d), NEG, jnp.float32)
        for w in range(NW):
            acc = jnp.maximum(acc, p_ref[w * b : (w + 1) * b, :])
        for i in range(nblk):
            seg = seg_ref[i]
            upd = jnp.maximum(acc, q_ref[i : i + 1, :])
            acc = jnp.where(segid == seg, upd, acc)
        o_ref[...] = acc

    return pl.pallas_call(
        body,
        in_specs=[
            pl.BlockSpec(memory_space=pltpu.SMEM),
            pl.BlockSpec((nblk, d), lambda: (0, 0)),
            pl.BlockSpec((NW * b, d), lambda: (0, 0)),
        ],
        out_specs=pl.BlockSpec((b, d), lambda: (0, 0)),
        out_shape=jax.ShapeDtypeStruct((b, d), jnp.float32),
    )(segmap, blockmax, partials_sc)


def kernel(flat, cu_seqlens):
    n, d = flat.shape
    b = cu_seqlens.shape[0] - 1
    nblk = n // GRAN
    assert n % (R_TC * NSTREAM) == 0 and d % LANES == 0

    cu = cu_seqlens.astype(jnp.int32)

    r0 = jnp.arange(nblk, dtype=jnp.int32) * GRAN
    inner = cu[1:b][None, :]
    s_first = jnp.sum(inner <= r0[:, None], axis=1, dtype=jnp.int32)
    s_last = jnp.sum(inner <= (r0 + GRAN - 1)[:, None], axis=1, dtype=jnp.int32)
    segmap = jnp.where(s_first == s_last, s_first, jnp.int32(b))

    cu_hi = cu[1:]
    pk = jnp.concatenate([cu[:b], cu_hi, jnp.repeat(cu_hi[: b - 1], LANES)])
    partials_sc = _sc_stage(flat, pk, d, b)
    blockmax = _tc_blocks(flat, n, d)
    return _tc_merge(
        partials_sc.reshape(NW * b, d), blockmax.reshape(nblk, d), segmap, b, d, nblk
    )
